# baseline (speedup 1.0000x reference)
import functools
import jax
import jax.numpy as jnp
from jax import lax
from jax.experimental import pallas as pl
from jax.experimental.pallas import tpu as pltpu
from jax.experimental.pallas import tpu_sc as plsc

N = 10000
E = 320000
F = 128
R = 8
B = 4

C = 64
NCHP = 320
NCH0 = 160
NCH1 = NCHP - NCH0
EPAD = 16 * NCHP * C
IPAD = 337408
NPAD = 10048
NGARB = NPAD - N
RPT = 640
RSTRIDE = 624
NRING = 4

_TCB = 1000
_ERB = EPAD // F
_ECB = _ERB // (N // _TCB)


def _project_body(wc_ref, x_ref, bases_ref, src_ref, typ_ref, xw_ref, gidx_ref):
    x = x_ref[...]
    for r in range(R):
        w = wc_ref[r, 0] * bases_ref[0]
        for b in range(1, B):
            w = w + wc_ref[r, b] * bases_ref[b]
        xw_ref[:, r * F:(r + 1) * F] = jnp.dot(
            x, w, preferred_element_type=jnp.float32)
    gidx_ref[...] = src_ref[...] * 8 + typ_ref[...]


def _finish_body(p0_ref, p1_ref, x_ref, lw_ref, bias_ref, g_ref, b_ref, out_ref):
    h = (p0_ref[...] + p1_ref[...]
         + jnp.dot(x_ref[...], lw_ref[...], preferred_element_type=jnp.float32)
         + bias_ref[...])
    h = jnp.where(h >= 0, h, 0.1 * h)
    m = jnp.mean(h, axis=1, keepdims=True)
    c = h - m
    v = jnp.mean(c * c, axis=1, keepdims=True)
    out_ref[...] = c * lax.rsqrt(v + 1e-5) * g_ref[...] + b_ref[...]


def _sc_body(xw_hbm, gidx_hbm, dst_hbm, zeros_hbm, out_hbm,
             gidx_v, d0, d1, d2, d3, rows0, rows1, rows2, rows3, agg_sh,
             semg0, semg1, semg2, semg3, semd0, semd1, semd2, semd3):
    cid = lax.axis_index("c")
    sid = lax.axis_index("s")

    pltpu.sync_copy(zeros_hbm.at[pl.ds(sid * RSTRIDE, RPT)],
                    agg_sh.at[pl.ds(sid * RSTRIDE, RPT)])
    plsc.subcore_barrier()

    ebase = (sid * NCHP + cid * NCH0) * C
    nch = NCH0 + cid * (NCH1 - NCH0)
    dbufs = (d0, d1, d2, d3)
    rbufs = (rows0, rows1, rows2, rows3)
    dsems = (semd0, semd1, semd2, semd3)
    gsems = (semg0, semg1, semg2, semg3)

    pltpu.sync_copy(gidx_hbm.at[pl.ds(ebase, NCH0 * C)], gidx_v)

    def gather(k, m):
        pltpu.async_copy(xw_hbm.at[gidx_v.at[pl.ds(k * C, C)]],
                         rbufs[m], gsems[m])

    def gather_wait(k, m):
        pltpu.make_async_copy(xw_hbm.at[gidx_v.at[pl.ds(k * C, C)]],
                              rbufs[m], gsems[m]).wait()

    def dst_load(k, m):
        pltpu.async_copy(dst_hbm.at[pl.ds(ebase + k * C, C)],
                         dbufs[m], dsems[m])

    def dst_wait(m):
        pltpu.make_async_copy(dst_hbm.at[pl.ds(ebase, C)],
                              dbufs[m], dsems[m]).wait()

    for m in range(NRING):
        dst_load(m, m)
        gather(m, m)

    def chunk_step(j, m, last):
        dst_wait(m)
        gather_wait(j, m)
        pltpu.sync_copy(rbufs[m], agg_sh.at[dbufs[m]], add=True)
        if not last:
            gather(j + NRING, m)
            dst_load(j + NRING, m)

    def body(t, carry):
        j0 = NRING * t
        for m in range(NRING):
            chunk_step(j0 + m, m, False)
        return carry

    lax.fori_loop(0, nch // NRING - 1, body, 0)
    j0 = nch - NRING
    for m in range(NRING):
        chunk_step(j0 + m, m, True)

    plsc.subcore_barrier()
    pltpu.sync_copy(agg_sh.at[pl.ds(sid * RSTRIDE, RPT)],
                    out_hbm.at[cid, pl.ds(sid * RSTRIDE, RPT)])


def kernel(node_feats, edge_index, edge_types, bases, w_comp, loop_weight,
           bias, ln_gamma, ln_beta):
    pad = EPAD - E
    src = jnp.pad(edge_index[0].astype(jnp.int32), (0, pad)).reshape(_ERB, F)
    typ = jnp.pad(edge_types.astype(jnp.int32), (0, pad)).reshape(_ERB, F)
    garb = N + (jnp.arange(pad, dtype=jnp.int32) % NGARB)
    dst_flat = jnp.concatenate([edge_index[1].astype(jnp.int32), garb])
    dst_flat = jnp.pad(dst_flat, (0, IPAD - EPAD))

    n_blocks = N // _TCB
    xw, gidx = pl.pallas_call(
        _project_body,
        grid=(n_blocks,),
        in_specs=[
            pl.BlockSpec(memory_space=pltpu.SMEM),
            pl.BlockSpec((_TCB, F), lambda i: (i, 0)),
            pl.BlockSpec((B, F, F), lambda i: (0, 0, 0)),
            pl.BlockSpec((_ECB, F), lambda i: (i, 0)),
            pl.BlockSpec((_ECB, F), lambda i: (i, 0)),
        ],
        out_specs=[
            pl.BlockSpec((_TCB, R * F), lambda i: (i, 0)),
            pl.BlockSpec((_ECB, F), lambda i: (i, 0)),
        ],
        out_shape=[
            jax.ShapeDtypeStruct((N, R * F), jnp.float32),
            jax.ShapeDtypeStruct((_ERB, F), jnp.int32),
        ],
    )(w_comp, node_feats, bases, src, typ)

    xw_rows = xw.reshape(N * R, F)
    gidx_flat = jnp.pad(gidx.reshape(-1), (0, IPAD - EPAD))
    zeros = jnp.zeros((N, F), jnp.float32)

    sc_scatter = functools.partial(
        pl.kernel,
        mesh=plsc.VectorSubcoreMesh(core_axis_name="c", subcore_axis_name="s"),
        out_type=jax.ShapeDtypeStruct((2, N, F), jnp.float32),
        scratch_types=[
            pltpu.VMEM((NCH0 * C,), jnp.int32),
            pltpu.VMEM((C,), jnp.int32),
            pltpu.VMEM((C,), jnp.int32),
            pltpu.VMEM((C,), jnp.int32),
            pltpu.VMEM((C,), jnp.int32),
            pltpu.VMEM((C, F), jnp.float32),
            pltpu.VMEM((C, F), jnp.float32),
            pltpu.VMEM((C, F), jnp.float32),
            pltpu.VMEM((C, F), jnp.float32),
            pltpu.VMEM_SHARED((NPAD, F), jnp.float32),
            pltpu.SemaphoreType.DMA,
            pltpu.SemaphoreType.DMA,
            pltpu.SemaphoreType.DMA,
            pltpu.SemaphoreType.DMA,
            pltpu.SemaphoreType.DMA,
            pltpu.SemaphoreType.DMA,
            pltpu.SemaphoreType.DMA,
            pltpu.SemaphoreType.DMA,
        ],
    )(_sc_body)
    partials = sc_scatter(xw_rows, gidx_flat, dst_flat, zeros)

    out = pl.pallas_call(
        _finish_body,
        grid=(n_blocks,),
        in_specs=[
            pl.BlockSpec((_TCB, F), lambda i: (i, 0)),
            pl.BlockSpec((_TCB, F), lambda i: (i, 0)),
            pl.BlockSpec((_TCB, F), lambda i: (i, 0)),
            pl.BlockSpec((F, F), lambda i: (0, 0)),
            pl.BlockSpec((1, F), lambda i: (0, 0)),
            pl.BlockSpec((1, F), lambda i: (0, 0)),
            pl.BlockSpec((1, F), lambda i: (0, 0)),
        ],
        out_specs=pl.BlockSpec((_TCB, F), lambda i: (i, 0)),
        out_shape=jax.ShapeDtypeStruct((N, F), jnp.float32),
    )(partials[0], partials[1], node_feats, loop_weight,
      bias.reshape(1, F), ln_gamma.reshape(1, F), ln_beta.reshape(1, F))
    return out


# speedup vs baseline: 1.0048x; 1.0048x over previous
import functools
import jax
import jax.numpy as jnp
from jax import lax
from jax.experimental import pallas as pl
from jax.experimental.pallas import tpu as pltpu
from jax.experimental.pallas import tpu_sc as plsc

N = 10000
E = 320000
F = 128
R = 8
B = 4

C = 64
NCHP = 320
NCH0 = 236
NCH1 = NCHP - NCH0
EPAD = 16 * NCHP * C
IPAD = 337408
NPAD = 10048
NGARB = NPAD - N
RPT = 640
RSTRIDE = 624
NRING = 4

_TCB = 1000
_ERB = EPAD // F
_ECB = _ERB // (N // _TCB)


def _project_body(wc_ref, x_ref, bases_ref, src_ref, typ_ref, xw_ref, gidx_ref):
    x = x_ref[...]
    for r in range(R):
        w = wc_ref[r, 0] * bases_ref[0]
        for b in range(1, B):
            w = w + wc_ref[r, b] * bases_ref[b]
        xw_ref[:, r * F:(r + 1) * F] = jnp.dot(
            x, w, preferred_element_type=jnp.float32)
    gidx_ref[...] = src_ref[...] * 8 + typ_ref[...]


def _finish_body(p0_ref, p1_ref, x_ref, lw_ref, bias_ref, g_ref, b_ref, out_ref):
    h = (p0_ref[...] + p1_ref[...]
         + jnp.dot(x_ref[...], lw_ref[...], preferred_element_type=jnp.float32)
         + bias_ref[...])
    h = jnp.where(h >= 0, h, 0.1 * h)
    m = jnp.mean(h, axis=1, keepdims=True)
    c = h - m
    v = jnp.mean(c * c, axis=1, keepdims=True)
    out_ref[...] = c * lax.rsqrt(v + 1e-5) * g_ref[...] + b_ref[...]


def _sc_body(xw_hbm, gidx_hbm, dst_hbm, zeros_hbm, out_hbm,
             gidx_v, d0, d1, d2, d3, rows0, rows1, rows2, rows3, agg_sh,
             semg0, semg1, semg2, semg3, semd0, semd1, semd2, semd3):
    cid = lax.axis_index("c")
    sid = lax.axis_index("s")

    pltpu.sync_copy(zeros_hbm.at[pl.ds(sid * RSTRIDE, RPT)],
                    agg_sh.at[pl.ds(sid * RSTRIDE, RPT)])
    plsc.subcore_barrier()

    ebase = (sid * NCHP + cid * NCH0) * C
    nch = NCH0 + cid * (NCH1 - NCH0)
    dbufs = (d0, d1, d2, d3)
    rbufs = (rows0, rows1, rows2, rows3)
    dsems = (semd0, semd1, semd2, semd3)
    gsems = (semg0, semg1, semg2, semg3)

    pltpu.sync_copy(gidx_hbm.at[pl.ds(ebase, NCH0 * C)], gidx_v)

    def gather(k, m):
        pltpu.async_copy(xw_hbm.at[gidx_v.at[pl.ds(k * C, C)]],
                         rbufs[m], gsems[m])

    def gather_wait(k, m):
        pltpu.make_async_copy(xw_hbm.at[gidx_v.at[pl.ds(k * C, C)]],
                              rbufs[m], gsems[m]).wait()

    def dst_load(k, m):
        pltpu.async_copy(dst_hbm.at[pl.ds(ebase + k * C, C)],
                         dbufs[m], dsems[m])

    def dst_wait(m):
        pltpu.make_async_copy(dst_hbm.at[pl.ds(ebase, C)],
                              dbufs[m], dsems[m]).wait()

    for m in range(NRING):
        dst_load(m, m)
        gather(m, m)

    def chunk_step(j, m, last):
        dst_wait(m)
        gather_wait(j, m)
        pltpu.sync_copy(rbufs[m], agg_sh.at[dbufs[m]], add=True)
        if not last:
            gather(j + NRING, m)
            dst_load(j + NRING, m)

    def body(t, carry):
        j0 = NRING * t
        for m in range(NRING):
            chunk_step(j0 + m, m, False)
        return carry

    lax.fori_loop(0, nch // NRING - 1, body, 0)
    j0 = nch - NRING
    for m in range(NRING):
        chunk_step(j0 + m, m, True)

    plsc.subcore_barrier()
    pltpu.sync_copy(agg_sh.at[pl.ds(sid * RSTRIDE, RPT)],
                    out_hbm.at[cid, pl.ds(sid * RSTRIDE, RPT)])


def kernel(node_feats, edge_index, edge_types, bases, w_comp, loop_weight,
           bias, ln_gamma, ln_beta):
    pad = EPAD - E
    src = jnp.pad(edge_index[0].astype(jnp.int32), (0, pad)).reshape(_ERB, F)
    typ = jnp.pad(edge_types.astype(jnp.int32), (0, pad)).reshape(_ERB, F)
    garb = N + (jnp.arange(pad, dtype=jnp.int32) % NGARB)
    dst_flat = jnp.concatenate([edge_index[1].astype(jnp.int32), garb])
    dst_flat = jnp.pad(dst_flat, (0, IPAD - EPAD))

    n_blocks = N // _TCB
    xw, gidx = pl.pallas_call(
        _project_body,
        grid=(n_blocks,),
        in_specs=[
            pl.BlockSpec(memory_space=pltpu.SMEM),
            pl.BlockSpec((_TCB, F), lambda i: (i, 0)),
            pl.BlockSpec((B, F, F), lambda i: (0, 0, 0)),
            pl.BlockSpec((_ECB, F), lambda i: (i, 0)),
            pl.BlockSpec((_ECB, F), lambda i: (i, 0)),
        ],
        out_specs=[
            pl.BlockSpec((_TCB, R * F), lambda i: (i, 0)),
            pl.BlockSpec((_ECB, F), lambda i: (i, 0)),
        ],
        out_shape=[
            jax.ShapeDtypeStruct((N, R * F), jnp.float32),
            jax.ShapeDtypeStruct((_ERB, F), jnp.int32),
        ],
    )(w_comp, node_feats, bases, src, typ)

    xw_rows = xw.reshape(N * R, F)
    gidx_flat = jnp.pad(gidx.reshape(-1), (0, IPAD - EPAD))
    zeros = jnp.zeros((N, F), jnp.float32)

    sc_scatter = functools.partial(
        pl.kernel,
        mesh=plsc.VectorSubcoreMesh(core_axis_name="c", subcore_axis_name="s"),
        out_type=jax.ShapeDtypeStruct((2, N, F), jnp.float32),
        scratch_types=[
            pltpu.VMEM((NCH0 * C,), jnp.int32),
            pltpu.VMEM((C,), jnp.int32),
            pltpu.VMEM((C,), jnp.int32),
            pltpu.VMEM((C,), jnp.int32),
            pltpu.VMEM((C,), jnp.int32),
            pltpu.VMEM((C, F), jnp.float32),
            pltpu.VMEM((C, F), jnp.float32),
            pltpu.VMEM((C, F), jnp.float32),
            pltpu.VMEM((C, F), jnp.float32),
            pltpu.VMEM_SHARED((NPAD, F), jnp.float32),
            pltpu.SemaphoreType.DMA,
            pltpu.SemaphoreType.DMA,
            pltpu.SemaphoreType.DMA,
            pltpu.SemaphoreType.DMA,
            pltpu.SemaphoreType.DMA,
            pltpu.SemaphoreType.DMA,
            pltpu.SemaphoreType.DMA,
            pltpu.SemaphoreType.DMA,
        ],
    )(_sc_body)
    partials = sc_scatter(xw_rows, gidx_flat, dst_flat, zeros)

    out = pl.pallas_call(
        _finish_body,
        grid=(n_blocks,),
        in_specs=[
            pl.BlockSpec((_TCB, F), lambda i: (i, 0)),
            pl.BlockSpec((_TCB, F), lambda i: (i, 0)),
            pl.BlockSpec((_TCB, F), lambda i: (i, 0)),
            pl.BlockSpec((F, F), lambda i: (0, 0)),
            pl.BlockSpec((1, F), lambda i: (0, 0)),
            pl.BlockSpec((1, F), lambda i: (0, 0)),
            pl.BlockSpec((1, F), lambda i: (0, 0)),
        ],
        out_specs=pl.BlockSpec((_TCB, F), lambda i: (i, 0)),
        out_shape=jax.ShapeDtypeStruct((N, F), jnp.float32),
    )(partials[0], partials[1], node_feats, loop_weight,
      bias.reshape(1, F), ln_gamma.reshape(1, F), ln_beta.reshape(1, F))
    return out
